# baseline (speedup 1.0000x reference)
import jax
import jax.numpy as jnp
from jax import lax
from jax.experimental import pallas as pl
from jax.experimental.pallas import tpu as pltpu

_H = 16
_W = 16
_HW = _H * _W
_CIN = 64
_CO = 32
_L = 4
_BN = 16
_BN2 = 16


def _fuse_bn(w, b, gamma, beta, mean, var, eps=1e-5):
    scale = gamma * lax.rsqrt(var + eps)
    return w * scale[:, None, None, None], (b - mean) * scale + beta


def _w_to_mat(w):
    cout, cin, kh, kw = w.shape
    return jnp.transpose(w, (0, 2, 3, 1)).reshape(cout, kh * kw * cin)


def _conv_kernel(x_ref, w_ref, bias_ref, o3_ref, ab_ref):
    bn = x_ref.shape[0]
    ln = bn * _HW
    xb = jnp.concatenate([x_ref[i] for i in range(bn)], axis=1)
    lane = lax.broadcasted_iota(jnp.int32, (1, ln), 1)
    p = lane % _HW
    hh = p // _W
    ww = p % _W

    w3 = w_ref[0:3 * _CO, 0:9 * _CIN]
    wl1 = w_ref[3 * _CO:4 * _CO, 0:_L * _CIN]
    w1l = w_ref[4 * _CO:5 * _CO, 0:_L * _CIN]
    b3 = bias_ref[0:3 * _CO]
    bl1 = bias_ref[3 * _CO:4 * _CO]
    b1l = bias_ref[4 * _CO:5 * _CO]

    def shifted(s):
        s = s % ln
        if s == 0:
            return xb
        return jnp.concatenate([xb[:, s:], xb[:, :s]], axis=1)

    zero = jnp.bfloat16(0)

    blocks = []
    for dh in range(3):
        for dw in range(3):
            sh = shifted((dh - 1) * _W + (dw - 1))
            valid = ((hh + (dh - 1) >= 0) & (hh + (dh - 1) < _H) &
                     (ww + (dw - 1) >= 0) & (ww + (dw - 1) < _W))
            blocks.append(jnp.where(valid, sh, zero))
    p3 = jnp.concatenate(blocks, axis=0)
    o3 = jnp.maximum(
        jnp.dot(w3, p3, preferred_element_type=jnp.float32) + b3, 0.0)

    blocks = []
    for dh in range(_L):
        blocks.append(jnp.where(hh + dh < _H, shifted(dh * _W), zero))
    pb = jnp.concatenate(blocks, axis=0)
    ob = jnp.maximum(
        jnp.dot(wl1, pb, preferred_element_type=jnp.float32) + bl1, 0.0)
    ob = jnp.where(hh < _H - _L + 1, ob, 0.0)

    blocks = []
    for dw in range(_L):
        blocks.append(jnp.where(ww + dw < _W, shifted(dw), zero))
    pa = jnp.concatenate(blocks, axis=0)
    oa = jnp.maximum(
        jnp.dot(w1l, pa, preferred_element_type=jnp.float32) + b1l, 0.0)

    ab = jnp.concatenate(
        [oa.astype(jnp.bfloat16), ob.astype(jnp.bfloat16)], axis=0)
    for i in range(bn):
        sl = slice(i * _HW, (i + 1) * _HW)
        o3_ref[i] = o3[:, sl]
        ab_ref[i] = ab[:, sl]


def _assemble_kernel(o3_ref, ab_ref, o_ref):
    o_ref[:, 0:2 * _CO, :] = o3_ref[:, 0:2 * _CO, :]
    o_ref[:, 3 * _CO:4 * _CO, :] = o3_ref[:, 2 * _CO:3 * _CO, :]
    bn = o3_ref.shape[0]
    for j in range(bn // 4):
        a4 = jnp.concatenate(
            [ab_ref[4 * j + i, 0:_CO, :] for i in range(4)], axis=0)
        b4 = jnp.concatenate(
            [ab_ref[4 * j + i, _CO:2 * _CO, :] for i in range(4)], axis=0)
        at3 = jnp.swapaxes(a4, 0, 1).reshape(_H, _W, 128)
        bt3 = jnp.swapaxes(b4, 0, 1).reshape(_H, _W, 128)
        bt3 = bt3.astype(jnp.float32)
        rows = []
        for hi in range(_H):
            ah = at3[hi].astype(jnp.float32)
            acc = ah[0:1, :] * bt3[0]
            for k in range(1, _H - _L + 1):
                acc = acc + ah[k:k + 1, :] * bt3[k]
            rows.append(acc)
        m = jnp.concatenate(rows, axis=0)
        mt = jnp.swapaxes(m, 0, 1)
        for i in range(4):
            o_ref[4 * j + i, 2 * _CO:3 * _CO, :] = (
                mt[_CO * i:_CO * (i + 1), :])


def kernel(x, b1x1_w, b1x1_b, b1x1_gamma, b1x1_beta, b1x1_mean, b1x1_var,
           b3x3_w, b3x3_b, b3x3_gamma, b3x3_beta, b3x3_mean, b3x3_var,
           bLx1_w, bLx1_b, bLx1_gamma, bLx1_beta, bLx1_mean, bLx1_var,
           b1xL_w, b1xL_b, b1xL_gamma, b1xL_beta, b1xL_mean, b1xL_var,
           bpool_w, bpool_b, bpool_gamma, bpool_beta, bpool_mean, bpool_var):
    n, cin, h, w = x.shape
    co = b1x1_w.shape[0]

    w1, c1 = _fuse_bn(b1x1_w, b1x1_b, b1x1_gamma, b1x1_beta, b1x1_mean, b1x1_var)
    w3, c3 = _fuse_bn(b3x3_w, b3x3_b, b3x3_gamma, b3x3_beta, b3x3_mean, b3x3_var)
    wl1, cl1 = _fuse_bn(bLx1_w, bLx1_b, bLx1_gamma, bLx1_beta, bLx1_mean, bLx1_var)
    w1l, c1l = _fuse_bn(b1xL_w, b1xL_b, b1xL_gamma, b1xL_beta, b1xL_mean, b1xL_var)
    wp, cp = _fuse_bn(bpool_w, bpool_b, bpool_gamma, bpool_beta, bpool_mean, bpool_var)

    w1_as3 = jnp.zeros((co, cin, 3, 3), jnp.float32).at[:, :, 1, 1].set(w1[:, :, 0, 0])
    wp_as3 = jnp.tile(wp / 9.0, (1, 1, 3, 3))
    w3_mat = _w_to_mat(jnp.concatenate([w1_as3, w3, wp_as3], axis=0))
    wl1_mat = jnp.transpose(wl1[:, :, :, 0], (0, 2, 1)).reshape(co, _L * cin)
    w1l_mat = jnp.transpose(w1l[:, :, 0, :], (0, 2, 1)).reshape(co, _L * cin)

    wpack = jnp.zeros((5 * co, 640), jnp.bfloat16)
    wpack = wpack.at[0:3 * co, 0:9 * cin].set(w3_mat.astype(jnp.bfloat16))
    wpack = wpack.at[3 * co:4 * co, 0:_L * cin].set(wl1_mat.astype(jnp.bfloat16))
    wpack = wpack.at[4 * co:5 * co, 0:_L * cin].set(w1l_mat.astype(jnp.bfloat16))
    bias = jnp.concatenate([c1, c3, cp, cl1, c1l]).reshape(5 * co, 1)

    xr = x.reshape(n, cin, _HW).astype(jnp.bfloat16)
    o3, ab = pl.pallas_call(
        _conv_kernel,
        out_shape=(
            jax.ShapeDtypeStruct((n, 3 * co, _HW), jnp.float32),
            jax.ShapeDtypeStruct((n, 2 * co, _HW), jnp.bfloat16),
        ),
        grid=(n // _BN,),
        in_specs=[
            pl.BlockSpec((_BN, cin, _HW), lambda i: (i, 0, 0)),
            pl.BlockSpec((5 * co, 640), lambda i: (0, 0)),
            pl.BlockSpec((5 * co, 1), lambda i: (0, 0)),
        ],
        out_specs=(
            pl.BlockSpec((_BN, 3 * co, _HW), lambda i: (i, 0, 0)),
            pl.BlockSpec((_BN, 2 * co, _HW), lambda i: (i, 0, 0)),
        ),
        compiler_params=pltpu.CompilerParams(dimension_semantics=("parallel",)),
    )(xr, wpack, bias)

    bn2 = _BN2
    out = pl.pallas_call(
        _assemble_kernel,
        out_shape=jax.ShapeDtypeStruct((n, 4 * co, _HW), jnp.float32),
        grid=(n // bn2,),
        in_specs=[
            pl.BlockSpec((bn2, 3 * co, _HW), lambda i: (i, 0, 0)),
            pl.BlockSpec((bn2, 2 * co, _HW), lambda i: (i, 0, 0)),
        ],
        out_specs=pl.BlockSpec((bn2, 4 * co, _HW), lambda i: (i, 0, 0)),
        compiler_params=pltpu.CompilerParams(dimension_semantics=("parallel",)),
    )(o3, ab)
    return out.reshape(n, 4 * co, h, w)


# speedup vs baseline: 1.0386x; 1.0386x over previous
import jax
import jax.numpy as jnp
from jax import lax
from jax.experimental import pallas as pl
from jax.experimental.pallas import tpu as pltpu

_H = 16
_W = 16
_HW = _H * _W
_CIN = 64
_CO = 32
_L = 4
_BN = 16
_BN2 = 16


def _fuse_bn(w, b, gamma, beta, mean, var, eps=1e-5):
    scale = gamma * lax.rsqrt(var + eps)
    return w * scale[:, None, None, None], (b - mean) * scale + beta


def _w_to_mat(w):
    cout, cin, kh, kw = w.shape
    return jnp.transpose(w, (0, 2, 3, 1)).reshape(cout, kh * kw * cin)


def _conv_kernel(x_ref, w_ref, bias_ref, o3_ref, ab_ref):
    bn = x_ref.shape[0]
    ln = bn * _HW
    xb = jnp.concatenate([x_ref[i] for i in range(bn)],
                         axis=1).astype(jnp.bfloat16)
    lane = lax.broadcasted_iota(jnp.int32, (1, ln), 1)
    p = lane % _HW
    hh = p // _W
    ww = p % _W

    w3 = w_ref[0:3 * _CO, 0:9 * _CIN]
    wl1 = w_ref[3 * _CO:4 * _CO, 0:_L * _CIN]
    w1l = w_ref[4 * _CO:5 * _CO, 0:_L * _CIN]
    b3 = bias_ref[0:3 * _CO]
    bl1 = bias_ref[3 * _CO:4 * _CO]
    b1l = bias_ref[4 * _CO:5 * _CO]

    def shifted(s):
        s = s % ln
        if s == 0:
            return xb
        return jnp.concatenate([xb[:, s:], xb[:, :s]], axis=1)

    zero = jnp.bfloat16(0)

    blocks = []
    for dh in range(3):
        for dw in range(3):
            sh = shifted((dh - 1) * _W + (dw - 1))
            valid = ((hh + (dh - 1) >= 0) & (hh + (dh - 1) < _H) &
                     (ww + (dw - 1) >= 0) & (ww + (dw - 1) < _W))
            blocks.append(jnp.where(valid, sh, zero))
    p3 = jnp.concatenate(blocks, axis=0)
    o3 = jnp.maximum(
        jnp.dot(w3, p3, preferred_element_type=jnp.float32) + b3, 0.0)

    blocks = []
    for dh in range(_L):
        blocks.append(jnp.where(hh + dh < _H, shifted(dh * _W), zero))
    pb = jnp.concatenate(blocks, axis=0)
    ob = jnp.maximum(
        jnp.dot(wl1, pb, preferred_element_type=jnp.float32) + bl1, 0.0)
    ob = jnp.where(hh < _H - _L + 1, ob, 0.0)

    blocks = []
    for dw in range(_L):
        blocks.append(jnp.where(ww + dw < _W, shifted(dw), zero))
    pa = jnp.concatenate(blocks, axis=0)
    oa = jnp.maximum(
        jnp.dot(w1l, pa, preferred_element_type=jnp.float32) + b1l, 0.0)

    ab = jnp.concatenate(
        [oa.astype(jnp.bfloat16), ob.astype(jnp.bfloat16)], axis=0)
    for i in range(bn):
        sl = slice(i * _HW, (i + 1) * _HW)
        o3_ref[i] = o3[:, sl]
        ab_ref[i] = ab[:, sl]


def _assemble_kernel(o3_ref, ab_ref, o_ref):
    o_ref[:, 0:2 * _CO, :] = o3_ref[:, 0:2 * _CO, :]
    o_ref[:, 3 * _CO:4 * _CO, :] = o3_ref[:, 2 * _CO:3 * _CO, :]
    bn = o3_ref.shape[0]
    for j in range(bn // 4):
        a4 = jnp.concatenate(
            [ab_ref[4 * j + i, 0:_CO, :] for i in range(4)], axis=0)
        b4 = jnp.concatenate(
            [ab_ref[4 * j + i, _CO:2 * _CO, :] for i in range(4)], axis=0)
        at3 = jnp.swapaxes(a4, 0, 1).reshape(_H, _W, 128)
        bt3 = jnp.swapaxes(b4, 0, 1).reshape(_H, _W, 128)
        bt3 = bt3.astype(jnp.float32)
        rows = []
        for hi in range(_H):
            ah = at3[hi].astype(jnp.float32)
            acc = ah[0:1, :] * bt3[0]
            for k in range(1, _H - _L + 1):
                acc = acc + ah[k:k + 1, :] * bt3[k]
            rows.append(acc)
        m = jnp.concatenate(rows, axis=0)
        mt = jnp.swapaxes(m, 0, 1)
        for i in range(4):
            o_ref[4 * j + i, 2 * _CO:3 * _CO, :] = (
                mt[_CO * i:_CO * (i + 1), :])


def kernel(x, b1x1_w, b1x1_b, b1x1_gamma, b1x1_beta, b1x1_mean, b1x1_var,
           b3x3_w, b3x3_b, b3x3_gamma, b3x3_beta, b3x3_mean, b3x3_var,
           bLx1_w, bLx1_b, bLx1_gamma, bLx1_beta, bLx1_mean, bLx1_var,
           b1xL_w, b1xL_b, b1xL_gamma, b1xL_beta, b1xL_mean, b1xL_var,
           bpool_w, bpool_b, bpool_gamma, bpool_beta, bpool_mean, bpool_var):
    n, cin, h, w = x.shape
    co = b1x1_w.shape[0]

    w1, c1 = _fuse_bn(b1x1_w, b1x1_b, b1x1_gamma, b1x1_beta, b1x1_mean, b1x1_var)
    w3, c3 = _fuse_bn(b3x3_w, b3x3_b, b3x3_gamma, b3x3_beta, b3x3_mean, b3x3_var)
    wl1, cl1 = _fuse_bn(bLx1_w, bLx1_b, bLx1_gamma, bLx1_beta, bLx1_mean, bLx1_var)
    w1l, c1l = _fuse_bn(b1xL_w, b1xL_b, b1xL_gamma, b1xL_beta, b1xL_mean, b1xL_var)
    wp, cp = _fuse_bn(bpool_w, bpool_b, bpool_gamma, bpool_beta, bpool_mean, bpool_var)

    w1_as3 = jnp.zeros((co, cin, 3, 3), jnp.float32).at[:, :, 1, 1].set(w1[:, :, 0, 0])
    wp_as3 = jnp.tile(wp / 9.0, (1, 1, 3, 3))
    w3_mat = _w_to_mat(jnp.concatenate([w1_as3, w3, wp_as3], axis=0))
    wl1_mat = jnp.transpose(wl1[:, :, :, 0], (0, 2, 1)).reshape(co, _L * cin)
    w1l_mat = jnp.transpose(w1l[:, :, 0, :], (0, 2, 1)).reshape(co, _L * cin)

    wpack = jnp.zeros((5 * co, 640), jnp.bfloat16)
    wpack = wpack.at[0:3 * co, 0:9 * cin].set(w3_mat.astype(jnp.bfloat16))
    wpack = wpack.at[3 * co:4 * co, 0:_L * cin].set(wl1_mat.astype(jnp.bfloat16))
    wpack = wpack.at[4 * co:5 * co, 0:_L * cin].set(w1l_mat.astype(jnp.bfloat16))
    bias = jnp.concatenate([c1, c3, cp, cl1, c1l]).reshape(5 * co, 1)

    xr = x.reshape(n, cin, _HW)
    o3, ab = pl.pallas_call(
        _conv_kernel,
        out_shape=(
            jax.ShapeDtypeStruct((n, 3 * co, _HW), jnp.float32),
            jax.ShapeDtypeStruct((n, 2 * co, _HW), jnp.bfloat16),
        ),
        grid=(n // _BN,),
        in_specs=[
            pl.BlockSpec((_BN, cin, _HW), lambda i: (i, 0, 0)),
            pl.BlockSpec((5 * co, 640), lambda i: (0, 0)),
            pl.BlockSpec((5 * co, 1), lambda i: (0, 0)),
        ],
        out_specs=(
            pl.BlockSpec((_BN, 3 * co, _HW), lambda i: (i, 0, 0)),
            pl.BlockSpec((_BN, 2 * co, _HW), lambda i: (i, 0, 0)),
        ),
        compiler_params=pltpu.CompilerParams(dimension_semantics=("parallel",)),
    )(xr, wpack, bias)

    bn2 = _BN2
    out = pl.pallas_call(
        _assemble_kernel,
        out_shape=jax.ShapeDtypeStruct((n, 4 * co, _HW), jnp.float32),
        grid=(n // bn2,),
        in_specs=[
            pl.BlockSpec((bn2, 3 * co, _HW), lambda i: (i, 0, 0)),
            pl.BlockSpec((bn2, 2 * co, _HW), lambda i: (i, 0, 0)),
        ],
        out_specs=pl.BlockSpec((bn2, 4 * co, _HW), lambda i: (i, 0, 0)),
        compiler_params=pltpu.CompilerParams(dimension_semantics=("parallel",)),
    )(o3, ab)
    return out.reshape(n, 4 * co, h, w)


# speedup vs baseline: 1.0542x; 1.0150x over previous
import jax
import jax.numpy as jnp
from jax import lax
from jax.experimental import pallas as pl
from jax.experimental.pallas import tpu as pltpu

_H = 16
_W = 16
_HW = _H * _W
_CIN = 64
_CO = 32
_L = 4
_BN = 16
_BN2 = 16


def _fuse_bn(w, b, gamma, beta, mean, var, eps=1e-5):
    scale = gamma * lax.rsqrt(var + eps)
    return w * scale[:, None, None, None], (b - mean) * scale + beta


def _w_to_mat(w):
    cout, cin, kh, kw = w.shape
    return jnp.transpose(w, (0, 2, 3, 1)).reshape(cout, kh * kw * cin)


def _conv_kernel(x_ref, w_ref, bias_ref, o3_ref, ab_ref):
    bn = x_ref.shape[0]
    ln = bn * _HW
    xb = jnp.concatenate([x_ref[i] for i in range(bn)],
                         axis=1).astype(jnp.bfloat16)
    lane = lax.broadcasted_iota(jnp.int32, (1, ln), 1)
    p = lane % _HW
    hh = p // _W
    ww = p % _W

    w3 = w_ref[0:3 * _CO, 0:9 * _CIN]
    wl1 = w_ref[3 * _CO:4 * _CO, 0:_L * _CIN]
    w1l = w_ref[4 * _CO:5 * _CO, 0:_L * _CIN]
    b3 = bias_ref[0:3 * _CO]
    bl1 = bias_ref[3 * _CO:4 * _CO]
    b1l = bias_ref[4 * _CO:5 * _CO]

    def shifted(s):
        s = s % ln
        if s == 0:
            return xb
        return jnp.concatenate([xb[:, s:], xb[:, :s]], axis=1)

    zero = jnp.bfloat16(0)

    blocks = []
    for dh in range(3):
        for dw in range(3):
            sh = shifted((dh - 1) * _W + (dw - 1))
            valid = ((hh + (dh - 1) >= 0) & (hh + (dh - 1) < _H) &
                     (ww + (dw - 1) >= 0) & (ww + (dw - 1) < _W))
            blocks.append(jnp.where(valid, sh, zero))
    p3 = jnp.concatenate(blocks, axis=0)
    o3 = jnp.maximum(
        jnp.dot(w3, p3, preferred_element_type=jnp.float32) + b3, 0.0)

    blocks = []
    for dh in range(_L):
        blocks.append(jnp.where(hh + dh < _H, shifted(dh * _W), zero))
    pb = jnp.concatenate(blocks, axis=0)
    ob = jnp.maximum(
        jnp.dot(wl1, pb, preferred_element_type=jnp.float32) + bl1, 0.0)
    ob = jnp.where(hh < _H - _L + 1, ob, 0.0)

    blocks = []
    for dw in range(_L):
        blocks.append(jnp.where(ww + dw < _W, shifted(dw), zero))
    pa = jnp.concatenate(blocks, axis=0)
    oa = jnp.maximum(
        jnp.dot(w1l, pa, preferred_element_type=jnp.float32) + b1l, 0.0)

    ab = jnp.concatenate(
        [oa.astype(jnp.bfloat16), ob.astype(jnp.bfloat16)], axis=0)
    o3b = o3.astype(jnp.bfloat16)
    for i in range(bn):
        sl = slice(i * _HW, (i + 1) * _HW)
        o3_ref[i] = o3b[:, sl]
        ab_ref[i] = ab[:, sl]


def _assemble_kernel(o3_ref, ab_ref, o_ref):
    o_ref[:, 0:2 * _CO, :] = o3_ref[:, 0:2 * _CO, :].astype(jnp.float32)
    o_ref[:, 3 * _CO:4 * _CO, :] = o3_ref[:, 2 * _CO:3 * _CO, :].astype(jnp.float32)
    bn = o3_ref.shape[0]
    for j in range(bn // 4):
        a4 = jnp.concatenate(
            [ab_ref[4 * j + i, 0:_CO, :] for i in range(4)], axis=0)
        b4 = jnp.concatenate(
            [ab_ref[4 * j + i, _CO:2 * _CO, :] for i in range(4)], axis=0)
        at3 = jnp.swapaxes(a4, 0, 1).reshape(_H, _W, 128)
        bt3 = jnp.swapaxes(b4, 0, 1).reshape(_H, _W, 128)
        bt3 = bt3.astype(jnp.float32)
        rows = []
        for hi in range(_H):
            ah = at3[hi].astype(jnp.float32)
            acc = ah[0:1, :] * bt3[0]
            for k in range(1, _H - _L + 1):
                acc = acc + ah[k:k + 1, :] * bt3[k]
            rows.append(acc)
        m = jnp.concatenate(rows, axis=0)
        mt = jnp.swapaxes(m, 0, 1)
        for i in range(4):
            o_ref[4 * j + i, 2 * _CO:3 * _CO, :] = (
                mt[_CO * i:_CO * (i + 1), :])


def kernel(x, b1x1_w, b1x1_b, b1x1_gamma, b1x1_beta, b1x1_mean, b1x1_var,
           b3x3_w, b3x3_b, b3x3_gamma, b3x3_beta, b3x3_mean, b3x3_var,
           bLx1_w, bLx1_b, bLx1_gamma, bLx1_beta, bLx1_mean, bLx1_var,
           b1xL_w, b1xL_b, b1xL_gamma, b1xL_beta, b1xL_mean, b1xL_var,
           bpool_w, bpool_b, bpool_gamma, bpool_beta, bpool_mean, bpool_var):
    n, cin, h, w = x.shape
    co = b1x1_w.shape[0]

    w1, c1 = _fuse_bn(b1x1_w, b1x1_b, b1x1_gamma, b1x1_beta, b1x1_mean, b1x1_var)
    w3, c3 = _fuse_bn(b3x3_w, b3x3_b, b3x3_gamma, b3x3_beta, b3x3_mean, b3x3_var)
    wl1, cl1 = _fuse_bn(bLx1_w, bLx1_b, bLx1_gamma, bLx1_beta, bLx1_mean, bLx1_var)
    w1l, c1l = _fuse_bn(b1xL_w, b1xL_b, b1xL_gamma, b1xL_beta, b1xL_mean, b1xL_var)
    wp, cp = _fuse_bn(bpool_w, bpool_b, bpool_gamma, bpool_beta, bpool_mean, bpool_var)

    w1_as3 = jnp.zeros((co, cin, 3, 3), jnp.float32).at[:, :, 1, 1].set(w1[:, :, 0, 0])
    wp_as3 = jnp.tile(wp / 9.0, (1, 1, 3, 3))
    w3_mat = _w_to_mat(jnp.concatenate([w1_as3, w3, wp_as3], axis=0))
    wl1_mat = jnp.transpose(wl1[:, :, :, 0], (0, 2, 1)).reshape(co, _L * cin)
    w1l_mat = jnp.transpose(w1l[:, :, 0, :], (0, 2, 1)).reshape(co, _L * cin)

    wpack = jnp.zeros((5 * co, 640), jnp.bfloat16)
    wpack = wpack.at[0:3 * co, 0:9 * cin].set(w3_mat.astype(jnp.bfloat16))
    wpack = wpack.at[3 * co:4 * co, 0:_L * cin].set(wl1_mat.astype(jnp.bfloat16))
    wpack = wpack.at[4 * co:5 * co, 0:_L * cin].set(w1l_mat.astype(jnp.bfloat16))
    bias = jnp.concatenate([c1, c3, cp, cl1, c1l]).reshape(5 * co, 1)

    xr = x.reshape(n, cin, _HW)
    o3, ab = pl.pallas_call(
        _conv_kernel,
        out_shape=(
            jax.ShapeDtypeStruct((n, 3 * co, _HW), jnp.bfloat16),
            jax.ShapeDtypeStruct((n, 2 * co, _HW), jnp.bfloat16),
        ),
        grid=(n // _BN,),
        in_specs=[
            pl.BlockSpec((_BN, cin, _HW), lambda i: (i, 0, 0)),
            pl.BlockSpec((5 * co, 640), lambda i: (0, 0)),
            pl.BlockSpec((5 * co, 1), lambda i: (0, 0)),
        ],
        out_specs=(
            pl.BlockSpec((_BN, 3 * co, _HW), lambda i: (i, 0, 0)),
            pl.BlockSpec((_BN, 2 * co, _HW), lambda i: (i, 0, 0)),
        ),
        compiler_params=pltpu.CompilerParams(dimension_semantics=("parallel",)),
    )(xr, wpack, bias)

    bn2 = _BN2
    out = pl.pallas_call(
        _assemble_kernel,
        out_shape=jax.ShapeDtypeStruct((n, 4 * co, _HW), jnp.float32),
        grid=(n // bn2,),
        in_specs=[
            pl.BlockSpec((bn2, 3 * co, _HW), lambda i: (i, 0, 0)),
            pl.BlockSpec((bn2, 2 * co, _HW), lambda i: (i, 0, 0)),
        ],
        out_specs=pl.BlockSpec((bn2, 4 * co, _HW), lambda i: (i, 0, 0)),
        compiler_params=pltpu.CompilerParams(dimension_semantics=("parallel",)),
    )(o3, ab)
    return out.reshape(n, 4 * co, h, w)


# speedup vs baseline: 1.0904x; 1.0343x over previous
import jax
import jax.numpy as jnp
from jax import lax
from jax.experimental import pallas as pl
from jax.experimental.pallas import tpu as pltpu

_H = 16
_W = 16
_HW = _H * _W
_CIN = 64
_CO = 32
_L = 4
_BN = 16
_BN2 = 32


def _fuse_bn(w, b, gamma, beta, mean, var, eps=1e-5):
    scale = gamma * lax.rsqrt(var + eps)
    return w * scale[:, None, None, None], (b - mean) * scale + beta


def _w_to_mat(w):
    cout, cin, kh, kw = w.shape
    return jnp.transpose(w, (0, 2, 3, 1)).reshape(cout, kh * kw * cin)


def _conv_kernel(x_ref, w_ref, bias_ref, o3_ref, ab_ref):
    bn = x_ref.shape[0]
    ln = bn * _HW
    xb = jnp.concatenate([x_ref[i] for i in range(bn)],
                         axis=1).astype(jnp.bfloat16)
    lane = lax.broadcasted_iota(jnp.int32, (1, ln), 1)
    p = lane % _HW
    hh = p // _W
    ww = p % _W

    w3 = w_ref[0:3 * _CO, 0:9 * _CIN]
    wl1 = w_ref[3 * _CO:4 * _CO, 0:_L * _CIN]
    w1l = w_ref[4 * _CO:5 * _CO, 0:_L * _CIN]
    b3 = bias_ref[0:3 * _CO]
    bl1 = bias_ref[3 * _CO:4 * _CO]
    b1l = bias_ref[4 * _CO:5 * _CO]

    def shifted(s):
        s = s % ln
        if s == 0:
            return xb
        return jnp.concatenate([xb[:, s:], xb[:, :s]], axis=1)

    zero = jnp.bfloat16(0)

    blocks = []
    for dh in range(3):
        for dw in range(3):
            sh = shifted((dh - 1) * _W + (dw - 1))
            valid = ((hh + (dh - 1) >= 0) & (hh + (dh - 1) < _H) &
                     (ww + (dw - 1) >= 0) & (ww + (dw - 1) < _W))
            blocks.append(jnp.where(valid, sh, zero))
    p3 = jnp.concatenate(blocks, axis=0)
    o3 = jnp.maximum(
        jnp.dot(w3, p3, preferred_element_type=jnp.float32) + b3, 0.0)

    blocks = []
    for dh in range(_L):
        blocks.append(jnp.where(hh + dh < _H, shifted(dh * _W), zero))
    pb = jnp.concatenate(blocks, axis=0)
    ob = jnp.maximum(
        jnp.dot(wl1, pb, preferred_element_type=jnp.float32) + bl1, 0.0)
    ob = jnp.where(hh < _H - _L + 1, ob, 0.0)

    blocks = []
    for dw in range(_L):
        blocks.append(jnp.where(ww + dw < _W, shifted(dw), zero))
    pa = jnp.concatenate(blocks, axis=0)
    oa = jnp.maximum(
        jnp.dot(w1l, pa, preferred_element_type=jnp.float32) + b1l, 0.0)

    ab = jnp.concatenate(
        [oa.astype(jnp.bfloat16), ob.astype(jnp.bfloat16)], axis=0)
    o3b = o3.astype(jnp.bfloat16)
    for i in range(bn):
        sl = slice(i * _HW, (i + 1) * _HW)
        o3_ref[i] = o3b[:, sl]
        ab_ref[i] = ab[:, sl]


def _assemble_kernel(o3_ref, ab_ref, o_ref):
    o_ref[:, 0:2 * _CO, :] = o3_ref[:, 0:2 * _CO, :].astype(jnp.float32)
    o_ref[:, 3 * _CO:4 * _CO, :] = o3_ref[:, 2 * _CO:3 * _CO, :].astype(jnp.float32)
    bn = o3_ref.shape[0]
    for j in range(bn // 4):
        a4 = jnp.concatenate(
            [ab_ref[4 * j + i, 0:_CO, :] for i in range(4)], axis=0)
        b4 = jnp.concatenate(
            [ab_ref[4 * j + i, _CO:2 * _CO, :] for i in range(4)], axis=0)
        at3 = jnp.swapaxes(a4, 0, 1).reshape(_H, _W, 128)
        bt3 = jnp.swapaxes(b4, 0, 1).reshape(_H, _W, 128)
        bt3 = bt3.astype(jnp.float32)
        rows = []
        for hi in range(_H):
            ah = at3[hi].astype(jnp.float32)
            acc = ah[0:1, :] * bt3[0]
            for k in range(1, _H - _L + 1):
                acc = acc + ah[k:k + 1, :] * bt3[k]
            rows.append(acc)
        m = jnp.concatenate(rows, axis=0)
        mt = jnp.swapaxes(m, 0, 1)
        for i in range(4):
            o_ref[4 * j + i, 2 * _CO:3 * _CO, :] = (
                mt[_CO * i:_CO * (i + 1), :])


def kernel(x, b1x1_w, b1x1_b, b1x1_gamma, b1x1_beta, b1x1_mean, b1x1_var,
           b3x3_w, b3x3_b, b3x3_gamma, b3x3_beta, b3x3_mean, b3x3_var,
           bLx1_w, bLx1_b, bLx1_gamma, bLx1_beta, bLx1_mean, bLx1_var,
           b1xL_w, b1xL_b, b1xL_gamma, b1xL_beta, b1xL_mean, b1xL_var,
           bpool_w, bpool_b, bpool_gamma, bpool_beta, bpool_mean, bpool_var):
    n, cin, h, w = x.shape
    co = b1x1_w.shape[0]

    w1, c1 = _fuse_bn(b1x1_w, b1x1_b, b1x1_gamma, b1x1_beta, b1x1_mean, b1x1_var)
    w3, c3 = _fuse_bn(b3x3_w, b3x3_b, b3x3_gamma, b3x3_beta, b3x3_mean, b3x3_var)
    wl1, cl1 = _fuse_bn(bLx1_w, bLx1_b, bLx1_gamma, bLx1_beta, bLx1_mean, bLx1_var)
    w1l, c1l = _fuse_bn(b1xL_w, b1xL_b, b1xL_gamma, b1xL_beta, b1xL_mean, b1xL_var)
    wp, cp = _fuse_bn(bpool_w, bpool_b, bpool_gamma, bpool_beta, bpool_mean, bpool_var)

    w1_as3 = jnp.zeros((co, cin, 3, 3), jnp.float32).at[:, :, 1, 1].set(w1[:, :, 0, 0])
    wp_as3 = jnp.tile(wp / 9.0, (1, 1, 3, 3))
    w3_mat = _w_to_mat(jnp.concatenate([w1_as3, w3, wp_as3], axis=0))
    wl1_mat = jnp.transpose(wl1[:, :, :, 0], (0, 2, 1)).reshape(co, _L * cin)
    w1l_mat = jnp.transpose(w1l[:, :, 0, :], (0, 2, 1)).reshape(co, _L * cin)

    wpack = jnp.zeros((5 * co, 640), jnp.bfloat16)
    wpack = wpack.at[0:3 * co, 0:9 * cin].set(w3_mat.astype(jnp.bfloat16))
    wpack = wpack.at[3 * co:4 * co, 0:_L * cin].set(wl1_mat.astype(jnp.bfloat16))
    wpack = wpack.at[4 * co:5 * co, 0:_L * cin].set(w1l_mat.astype(jnp.bfloat16))
    bias = jnp.concatenate([c1, c3, cp, cl1, c1l]).reshape(5 * co, 1)

    xr = x.reshape(n, cin, _HW)
    o3, ab = pl.pallas_call(
        _conv_kernel,
        out_shape=(
            jax.ShapeDtypeStruct((n, 3 * co, _HW), jnp.bfloat16),
            jax.ShapeDtypeStruct((n, 2 * co, _HW), jnp.bfloat16),
        ),
        grid=(n // _BN,),
        in_specs=[
            pl.BlockSpec((_BN, cin, _HW), lambda i: (i, 0, 0)),
            pl.BlockSpec((5 * co, 640), lambda i: (0, 0)),
            pl.BlockSpec((5 * co, 1), lambda i: (0, 0)),
        ],
        out_specs=(
            pl.BlockSpec((_BN, 3 * co, _HW), lambda i: (i, 0, 0)),
            pl.BlockSpec((_BN, 2 * co, _HW), lambda i: (i, 0, 0)),
        ),
        compiler_params=pltpu.CompilerParams(dimension_semantics=("parallel",)),
    )(xr, wpack, bias)

    bn2 = _BN2
    out = pl.pallas_call(
        _assemble_kernel,
        out_shape=jax.ShapeDtypeStruct((n, 4 * co, _HW), jnp.float32),
        grid=(n // bn2,),
        in_specs=[
            pl.BlockSpec((bn2, 3 * co, _HW), lambda i: (i, 0, 0)),
            pl.BlockSpec((bn2, 2 * co, _HW), lambda i: (i, 0, 0)),
        ],
        out_specs=pl.BlockSpec((bn2, 4 * co, _HW), lambda i: (i, 0, 0)),
        compiler_params=pltpu.CompilerParams(dimension_semantics=("parallel",)),
    )(o3, ab)
    return out.reshape(n, 4 * co, h, w)


# speedup vs baseline: 1.1040x; 1.0125x over previous
import jax
import jax.numpy as jnp
from jax import lax
from jax.experimental import pallas as pl
from jax.experimental.pallas import tpu as pltpu

_H = 16
_W = 16
_HW = _H * _W
_CIN = 64
_CO = 32
_L = 4
_BN = 16
_BN2 = 64


def _fuse_bn(w, b, gamma, beta, mean, var, eps=1e-5):
    scale = gamma * lax.rsqrt(var + eps)
    return w * scale[:, None, None, None], (b - mean) * scale + beta


def _w_to_mat(w):
    cout, cin, kh, kw = w.shape
    return jnp.transpose(w, (0, 2, 3, 1)).reshape(cout, kh * kw * cin)


def _conv_kernel(x_ref, w_ref, bias_ref, o3_ref, ab_ref):
    bn = x_ref.shape[0]
    ln = bn * _HW
    xb = jnp.concatenate([x_ref[i] for i in range(bn)],
                         axis=1).astype(jnp.bfloat16)
    lane = lax.broadcasted_iota(jnp.int32, (1, ln), 1)
    p = lane % _HW
    hh = p // _W
    ww = p % _W

    w3 = w_ref[0:3 * _CO, 0:9 * _CIN]
    wl1 = w_ref[3 * _CO:4 * _CO, 0:_L * _CIN]
    w1l = w_ref[4 * _CO:5 * _CO, 0:_L * _CIN]
    b3 = bias_ref[0:3 * _CO]
    bl1 = bias_ref[3 * _CO:4 * _CO]
    b1l = bias_ref[4 * _CO:5 * _CO]

    def shifted(s):
        s = s % ln
        if s == 0:
            return xb
        return jnp.concatenate([xb[:, s:], xb[:, :s]], axis=1)

    zero = jnp.bfloat16(0)

    blocks = []
    for dh in range(3):
        for dw in range(3):
            sh = shifted((dh - 1) * _W + (dw - 1))
            valid = ((hh + (dh - 1) >= 0) & (hh + (dh - 1) < _H) &
                     (ww + (dw - 1) >= 0) & (ww + (dw - 1) < _W))
            blocks.append(jnp.where(valid, sh, zero))
    p3 = jnp.concatenate(blocks, axis=0)
    o3 = jnp.maximum(
        jnp.dot(w3, p3, preferred_element_type=jnp.float32) + b3, 0.0)

    blocks = []
    for dh in range(_L):
        blocks.append(jnp.where(hh + dh < _H, shifted(dh * _W), zero))
    pb = jnp.concatenate(blocks, axis=0)
    ob = jnp.maximum(
        jnp.dot(wl1, pb, preferred_element_type=jnp.float32) + bl1, 0.0)
    ob = jnp.where(hh < _H - _L + 1, ob, 0.0)

    blocks = []
    for dw in range(_L):
        blocks.append(jnp.where(ww + dw < _W, shifted(dw), zero))
    pa = jnp.concatenate(blocks, axis=0)
    oa = jnp.maximum(
        jnp.dot(w1l, pa, preferred_element_type=jnp.float32) + b1l, 0.0)

    ab = jnp.concatenate(
        [oa.astype(jnp.bfloat16), ob.astype(jnp.bfloat16)], axis=0)
    o3b = o3.astype(jnp.bfloat16)
    for i in range(bn):
        sl = slice(i * _HW, (i + 1) * _HW)
        o3_ref[i] = o3b[:, sl]
        ab_ref[i] = ab[:, sl]


def _assemble_kernel(o3_ref, ab_ref, o_ref):
    o_ref[:, 0:2 * _CO, :] = o3_ref[:, 0:2 * _CO, :].astype(jnp.float32)
    o_ref[:, 3 * _CO:4 * _CO, :] = o3_ref[:, 2 * _CO:3 * _CO, :].astype(jnp.float32)
    bn = o3_ref.shape[0]
    for j in range(bn // 4):
        a4 = jnp.concatenate(
            [ab_ref[4 * j + i, 0:_CO, :] for i in range(4)], axis=0)
        b4 = jnp.concatenate(
            [ab_ref[4 * j + i, _CO:2 * _CO, :] for i in range(4)], axis=0)
        at3 = jnp.swapaxes(a4, 0, 1).reshape(_H, _W, 128)
        bt3 = jnp.swapaxes(b4, 0, 1).reshape(_H, _W, 128)
        bt3 = bt3.astype(jnp.float32)
        rows = []
        for hi in range(_H):
            ah = at3[hi].astype(jnp.float32)
            acc = ah[0:1, :] * bt3[0]
            for k in range(1, _H - _L + 1):
                acc = acc + ah[k:k + 1, :] * bt3[k]
            rows.append(acc)
        m = jnp.concatenate(rows, axis=0)
        mt = jnp.swapaxes(m, 0, 1)
        for i in range(4):
            o_ref[4 * j + i, 2 * _CO:3 * _CO, :] = (
                mt[_CO * i:_CO * (i + 1), :])


def kernel(x, b1x1_w, b1x1_b, b1x1_gamma, b1x1_beta, b1x1_mean, b1x1_var,
           b3x3_w, b3x3_b, b3x3_gamma, b3x3_beta, b3x3_mean, b3x3_var,
           bLx1_w, bLx1_b, bLx1_gamma, bLx1_beta, bLx1_mean, bLx1_var,
           b1xL_w, b1xL_b, b1xL_gamma, b1xL_beta, b1xL_mean, b1xL_var,
           bpool_w, bpool_b, bpool_gamma, bpool_beta, bpool_mean, bpool_var):
    n, cin, h, w = x.shape
    co = b1x1_w.shape[0]

    w1, c1 = _fuse_bn(b1x1_w, b1x1_b, b1x1_gamma, b1x1_beta, b1x1_mean, b1x1_var)
    w3, c3 = _fuse_bn(b3x3_w, b3x3_b, b3x3_gamma, b3x3_beta, b3x3_mean, b3x3_var)
    wl1, cl1 = _fuse_bn(bLx1_w, bLx1_b, bLx1_gamma, bLx1_beta, bLx1_mean, bLx1_var)
    w1l, c1l = _fuse_bn(b1xL_w, b1xL_b, b1xL_gamma, b1xL_beta, b1xL_mean, b1xL_var)
    wp, cp = _fuse_bn(bpool_w, bpool_b, bpool_gamma, bpool_beta, bpool_mean, bpool_var)

    w1_as3 = jnp.zeros((co, cin, 3, 3), jnp.float32).at[:, :, 1, 1].set(w1[:, :, 0, 0])
    wp_as3 = jnp.tile(wp / 9.0, (1, 1, 3, 3))
    w3_mat = _w_to_mat(jnp.concatenate([w1_as3, w3, wp_as3], axis=0))
    wl1_mat = jnp.transpose(wl1[:, :, :, 0], (0, 2, 1)).reshape(co, _L * cin)
    w1l_mat = jnp.transpose(w1l[:, :, 0, :], (0, 2, 1)).reshape(co, _L * cin)

    wpack = jnp.zeros((5 * co, 640), jnp.bfloat16)
    wpack = wpack.at[0:3 * co, 0:9 * cin].set(w3_mat.astype(jnp.bfloat16))
    wpack = wpack.at[3 * co:4 * co, 0:_L * cin].set(wl1_mat.astype(jnp.bfloat16))
    wpack = wpack.at[4 * co:5 * co, 0:_L * cin].set(w1l_mat.astype(jnp.bfloat16))
    bias = jnp.concatenate([c1, c3, cp, cl1, c1l]).reshape(5 * co, 1)

    xr = x.reshape(n, cin, _HW)
    o3, ab = pl.pallas_call(
        _conv_kernel,
        out_shape=(
            jax.ShapeDtypeStruct((n, 3 * co, _HW), jnp.bfloat16),
            jax.ShapeDtypeStruct((n, 2 * co, _HW), jnp.bfloat16),
        ),
        grid=(n // _BN,),
        in_specs=[
            pl.BlockSpec((_BN, cin, _HW), lambda i: (i, 0, 0)),
            pl.BlockSpec((5 * co, 640), lambda i: (0, 0)),
            pl.BlockSpec((5 * co, 1), lambda i: (0, 0)),
        ],
        out_specs=(
            pl.BlockSpec((_BN, 3 * co, _HW), lambda i: (i, 0, 0)),
            pl.BlockSpec((_BN, 2 * co, _HW), lambda i: (i, 0, 0)),
        ),
        compiler_params=pltpu.CompilerParams(dimension_semantics=("parallel",)),
    )(xr, wpack, bias)

    bn2 = _BN2
    out = pl.pallas_call(
        _assemble_kernel,
        out_shape=jax.ShapeDtypeStruct((n, 4 * co, _HW), jnp.float32),
        grid=(n // bn2,),
        in_specs=[
            pl.BlockSpec((bn2, 3 * co, _HW), lambda i: (i, 0, 0)),
            pl.BlockSpec((bn2, 2 * co, _HW), lambda i: (i, 0, 0)),
        ],
        out_specs=pl.BlockSpec((bn2, 4 * co, _HW), lambda i: (i, 0, 0)),
        compiler_params=pltpu.CompilerParams(dimension_semantics=("parallel",)),
    )(o3, ab)
    return out.reshape(n, 4 * co, h, w)


# speedup vs baseline: 1.1812x; 1.0699x over previous
import jax
import jax.numpy as jnp
from jax import lax
from jax.experimental import pallas as pl
from jax.experimental.pallas import tpu as pltpu

_H = 16
_W = 16
_HW = _H * _W
_CIN = 64
_CO = 32
_L = 4
_BN = 16


def _fuse_bn(w, b, gamma, beta, mean, var, eps=1e-5):
    scale = gamma * lax.rsqrt(var + eps)
    return w * scale[:, None, None, None], (b - mean) * scale + beta


def _w_to_mat(w):
    cout, cin, kh, kw = w.shape
    return jnp.transpose(w, (0, 2, 3, 1)).reshape(cout, kh * kw * cin)


def _block_kernel(x_ref, w_ref, bias_ref, o_ref):
    bn = x_ref.shape[0]
    ln = bn * _HW
    xb = jnp.concatenate([x_ref[i] for i in range(bn)],
                         axis=1).astype(jnp.bfloat16)
    lane = lax.broadcasted_iota(jnp.int32, (1, ln), 1)
    p = lane % _HW
    hh = p // _W
    ww = p % _W

    w3 = w_ref[0:3 * _CO, 0:9 * _CIN]
    wl1 = w_ref[3 * _CO:4 * _CO, 0:_L * _CIN]
    w1l = w_ref[4 * _CO:5 * _CO, 0:_L * _CIN]
    b3 = bias_ref[0:3 * _CO]
    bl1 = bias_ref[3 * _CO:4 * _CO]
    b1l = bias_ref[4 * _CO:5 * _CO]

    def shifted(s):
        s = s % ln
        if s == 0:
            return xb
        return jnp.concatenate([xb[:, s:], xb[:, :s]], axis=1)

    zero = jnp.bfloat16(0)

    blocks = []
    for dh in range(3):
        for dw in range(3):
            sh = shifted((dh - 1) * _W + (dw - 1))
            valid = ((hh + (dh - 1) >= 0) & (hh + (dh - 1) < _H) &
                     (ww + (dw - 1) >= 0) & (ww + (dw - 1) < _W))
            blocks.append(jnp.where(valid, sh, zero))
    p3 = jnp.concatenate(blocks, axis=0)
    o3 = jnp.maximum(
        jnp.dot(w3, p3, preferred_element_type=jnp.float32) + b3, 0.0)

    blocks = []
    for dh in range(_L):
        blocks.append(jnp.where(hh + dh < _H, shifted(dh * _W), zero))
    pb = jnp.concatenate(blocks, axis=0)
    ob = jnp.maximum(
        jnp.dot(wl1, pb, preferred_element_type=jnp.float32) + bl1, 0.0)
    ob = jnp.where(hh < _H - _L + 1, ob, 0.0)

    blocks = []
    for dw in range(_L):
        blocks.append(jnp.where(ww + dw < _W, shifted(dw), zero))
    pa = jnp.concatenate(blocks, axis=0)
    oa = jnp.maximum(
        jnp.dot(w1l, pa, preferred_element_type=jnp.float32) + b1l, 0.0)

    oab = oa.astype(jnp.bfloat16)
    obb = ob.astype(jnp.bfloat16)

    for i in range(bn):
        sl = slice(i * _HW, (i + 1) * _HW)
        o_ref[i, 0:2 * _CO, :] = o3[0:2 * _CO, sl]
        o_ref[i, 3 * _CO:4 * _CO, :] = o3[2 * _CO:3 * _CO, sl]

    for j in range(bn // 4):
        a4 = jnp.concatenate(
            [oab[:, (4 * j + i) * _HW:(4 * j + i + 1) * _HW]
             for i in range(4)], axis=0)
        b4 = jnp.concatenate(
            [obb[:, (4 * j + i) * _HW:(4 * j + i + 1) * _HW]
             for i in range(4)], axis=0)
        at3 = jnp.swapaxes(a4, 0, 1).reshape(_H, _W, 128)
        bt3 = jnp.swapaxes(b4, 0, 1).reshape(_H, _W, 128)
        bt3 = bt3.astype(jnp.float32)
        rows = []
        for hi in range(_H):
            ah = at3[hi].astype(jnp.float32)
            acc = ah[0:1, :] * bt3[0]
            for k in range(1, _H - _L + 1):
                acc = acc + ah[k:k + 1, :] * bt3[k]
            rows.append(acc)
        m = jnp.concatenate(rows, axis=0)
        mt = jnp.swapaxes(m, 0, 1)
        for i in range(4):
            o_ref[4 * j + i, 2 * _CO:3 * _CO, :] = (
                mt[_CO * i:_CO * (i + 1), :])


def kernel(x, b1x1_w, b1x1_b, b1x1_gamma, b1x1_beta, b1x1_mean, b1x1_var,
           b3x3_w, b3x3_b, b3x3_gamma, b3x3_beta, b3x3_mean, b3x3_var,
           bLx1_w, bLx1_b, bLx1_gamma, bLx1_beta, bLx1_mean, bLx1_var,
           b1xL_w, b1xL_b, b1xL_gamma, b1xL_beta, b1xL_mean, b1xL_var,
           bpool_w, bpool_b, bpool_gamma, bpool_beta, bpool_mean, bpool_var):
    n, cin, h, w = x.shape
    co = b1x1_w.shape[0]

    w1, c1 = _fuse_bn(b1x1_w, b1x1_b, b1x1_gamma, b1x1_beta, b1x1_mean, b1x1_var)
    w3, c3 = _fuse_bn(b3x3_w, b3x3_b, b3x3_gamma, b3x3_beta, b3x3_mean, b3x3_var)
    wl1, cl1 = _fuse_bn(bLx1_w, bLx1_b, bLx1_gamma, bLx1_beta, bLx1_mean, bLx1_var)
    w1l, c1l = _fuse_bn(b1xL_w, b1xL_b, b1xL_gamma, b1xL_beta, b1xL_mean, b1xL_var)
    wp, cp = _fuse_bn(bpool_w, bpool_b, bpool_gamma, bpool_beta, bpool_mean, bpool_var)

    w1_as3 = jnp.zeros((co, cin, 3, 3), jnp.float32).at[:, :, 1, 1].set(w1[:, :, 0, 0])
    wp_as3 = jnp.tile(wp / 9.0, (1, 1, 3, 3))
    w3_mat = _w_to_mat(jnp.concatenate([w1_as3, w3, wp_as3], axis=0))
    wl1_mat = jnp.transpose(wl1[:, :, :, 0], (0, 2, 1)).reshape(co, _L * cin)
    w1l_mat = jnp.transpose(w1l[:, :, 0, :], (0, 2, 1)).reshape(co, _L * cin)

    wpack = jnp.zeros((5 * co, 640), jnp.bfloat16)
    wpack = wpack.at[0:3 * co, 0:9 * cin].set(w3_mat.astype(jnp.bfloat16))
    wpack = wpack.at[3 * co:4 * co, 0:_L * cin].set(wl1_mat.astype(jnp.bfloat16))
    wpack = wpack.at[4 * co:5 * co, 0:_L * cin].set(w1l_mat.astype(jnp.bfloat16))
    bias = jnp.concatenate([c1, c3, cp, cl1, c1l]).reshape(5 * co, 1)

    xr = x.reshape(n, cin, _HW)
    out = pl.pallas_call(
        _block_kernel,
        out_shape=jax.ShapeDtypeStruct((n, 4 * co, _HW), jnp.float32),
        grid=(n // _BN,),
        in_specs=[
            pl.BlockSpec((_BN, cin, _HW), lambda i: (i, 0, 0)),
            pl.BlockSpec((5 * co, 640), lambda i: (0, 0)),
            pl.BlockSpec((5 * co, 1), lambda i: (0, 0)),
        ],
        out_specs=pl.BlockSpec((_BN, 4 * co, _HW), lambda i: (i, 0, 0)),
        compiler_params=pltpu.CompilerParams(dimension_semantics=("parallel",)),
    )(xr, wpack, bias)
    return out.reshape(n, 4 * co, h, w)
